# baseline (speedup 1.0000x reference)
import functools

import jax
import jax.numpy as jnp
from jax import lax
from jax.experimental import pallas as pl
from jax.experimental.pallas import tpu as pltpu
from jax.experimental.pallas import tpu_sc as plsc

NUM_KNOTS = 20
X_MIN = -5.0
X_MAX = 5.0
_DX = (X_MAX - X_MIN) / (NUM_KNOTS - 1)

_NC = 2
_NS = 16
_NW = _NC * _NS
_L = 16

_CHUNK = 16384


def _spline_body(x_hbm, values_hbm, out_hbm, vals_v, t3, t2, t1, t0, xb, ob):
    wid = lax.axis_index("s") * _NC + lax.axis_index("c")
    n_total = x_hbm.shape[0]
    per_w = n_total // _NW
    base = wid * per_w

    pltpu.sync_copy(values_hbm, vals_v)
    for t in range(2):
        k = lax.iota(jnp.int32, _L) + (t * _L)
        kc = jnp.minimum(k, NUM_KNOTS - 1)
        km1 = jnp.clip(k - 1, 0, NUM_KNOTS - 1)
        kp1 = jnp.minimum(k + 1, NUM_KNOTS - 1)
        kp2 = jnp.minimum(k + 2, NUM_KNOTS - 1)
        v0 = plsc.load_gather(vals_v, [km1])
        v1 = plsc.load_gather(vals_v, [kc])
        v2 = plsc.load_gather(vals_v, [kp1])
        v3 = plsc.load_gather(vals_v, [kp2])
        t3[pl.ds(t * _L, _L)] = 0.5 * (-v0 + 3.0 * v1 - 3.0 * v2 + v3)
        t2[pl.ds(t * _L, _L)] = 0.5 * (2.0 * v0 - 5.0 * v1 + 4.0 * v2 - v3)
        t1[pl.ds(t * _L, _L)] = 0.5 * (v2 - v0)
        t0[pl.ds(t * _L, _L)] = v1

    inv_dx = jnp.float32(1.0 / _DX)

    def chunk_body(ci, _):
        off = base + ci * _CHUNK
        pltpu.sync_copy(x_hbm.at[pl.ds(off, _CHUNK)], xb)

        def vec_body(j, __):
            xv = xb[pl.ds(j * _L, _L)]
            xc = jnp.minimum(jnp.maximum(xv, jnp.float32(X_MIN)),
                             jnp.float32(X_MAX))
            u = (xc - jnp.float32(X_MIN)) * inv_dx
            i = u.astype(jnp.int32)
            f = u - i.astype(jnp.float32)
            c3 = plsc.load_gather(t3, [i])
            c2 = plsc.load_gather(t2, [i])
            c1 = plsc.load_gather(t1, [i])
            c0 = plsc.load_gather(t0, [i])
            ob[pl.ds(j * _L, _L)] = ((c3 * f + c2) * f + c1) * f + c0
            return 0

        lax.fori_loop(0, _CHUNK // _L, vec_body, 0)
        pltpu.sync_copy(ob, out_hbm.at[pl.ds(off, _CHUNK)])
        return 0

    lax.fori_loop(0, per_w // _CHUNK, chunk_body, 0)


def kernel(x, values):
    n_total = x.size
    mesh = plsc.VectorSubcoreMesh(core_axis_name="c", subcore_axis_name="s")
    vals_pad = jnp.pad(values, (0, 32 - NUM_KNOTS))
    run = functools.partial(
        pl.kernel,
        mesh=mesh,
        compiler_params=pltpu.CompilerParams(needs_layout_passes=False),
        out_type=jax.ShapeDtypeStruct((n_total,), jnp.float32),
        scratch_types=[
            pltpu.VMEM((32,), jnp.float32),
            pltpu.VMEM((32,), jnp.float32),
            pltpu.VMEM((32,), jnp.float32),
            pltpu.VMEM((32,), jnp.float32),
            pltpu.VMEM((32,), jnp.float32),
            pltpu.VMEM((_CHUNK,), jnp.float32),
            pltpu.VMEM((_CHUNK,), jnp.float32),
        ],
    )(_spline_body)
    out_flat = run(x.reshape(-1), vals_pad)
    return out_flat.reshape(x.shape)


# speedup vs baseline: 1.2373x; 1.2373x over previous
import functools

import jax
import jax.numpy as jnp
from jax import lax
from jax.experimental import pallas as pl
from jax.experimental.pallas import tpu as pltpu
from jax.experimental.pallas import tpu_sc as plsc

NUM_KNOTS = 20
X_MIN = -5.0
X_MAX = 5.0
_DX = (X_MAX - X_MIN) / (NUM_KNOTS - 1)

_NC = 2
_NS = 16
_NW = _NC * _NS
_L = 16

_CHUNK = 16384


_UNROLL = 4


def _spline_body(x_hbm, values_hbm, out_hbm, vals_v, t3, t2, t1, t0,
                 xb0, xb1, ob0, ob1, si0, si1, so0, so1):
    wid = lax.axis_index("s") * _NC + lax.axis_index("c")
    n_total = x_hbm.shape[0]
    per_w = n_total // _NW
    base = wid * per_w

    pltpu.sync_copy(values_hbm, vals_v)
    for t in range(2):
        k = lax.iota(jnp.int32, _L) + (t * _L)
        kc = jnp.minimum(k, NUM_KNOTS - 1)
        km1 = jnp.clip(k - 1, 0, NUM_KNOTS - 1)
        kp1 = jnp.minimum(k + 1, NUM_KNOTS - 1)
        kp2 = jnp.minimum(k + 2, NUM_KNOTS - 1)
        v0 = plsc.load_gather(vals_v, [km1])
        v1 = plsc.load_gather(vals_v, [kc])
        v2 = plsc.load_gather(vals_v, [kp1])
        v3 = plsc.load_gather(vals_v, [kp2])
        t3[pl.ds(t * _L, _L)] = 0.5 * (-v0 + 3.0 * v1 - 3.0 * v2 + v3)
        t2[pl.ds(t * _L, _L)] = 0.5 * (2.0 * v0 - 5.0 * v1 + 4.0 * v2 - v3)
        t1[pl.ds(t * _L, _L)] = 0.5 * (v2 - v0)
        t0[pl.ds(t * _L, _L)] = v1

    inv_dx = jnp.float32(1.0 / _DX)
    nch = per_w // _CHUNK

    def compute_chunk(xb, ob):
        def vec_body(j, __):
            s0 = j * (_UNROLL * _L)
            for r in range(_UNROLL):
                s = s0 + r * _L
                xv = xb[pl.ds(s, _L)]
                xc = jnp.minimum(jnp.maximum(xv, jnp.float32(X_MIN)),
                                 jnp.float32(X_MAX))
                u = (xc - jnp.float32(X_MIN)) * inv_dx
                i = u.astype(jnp.int32)
                f = u - i.astype(jnp.float32)
                c3 = plsc.load_gather(t3, [i])
                c2 = plsc.load_gather(t2, [i])
                c1 = plsc.load_gather(t1, [i])
                c0 = plsc.load_gather(t0, [i])
                ob[pl.ds(s, _L)] = ((c3 * f + c2) * f + c1) * f + c0
            return 0

        lax.fori_loop(0, _CHUNK // (_UNROLL * _L), vec_body, 0)

    pltpu.async_copy(x_hbm.at[pl.ds(base, _CHUNK)], xb0, si0)
    pltpu.async_copy(x_hbm.at[pl.ds(base + _CHUNK, _CHUNK)], xb1, si1)

    bufs = ((xb0, ob0, si0, so0), (xb1, ob1, si1, so1))

    def pair_body(g2, _):
        for b, (xb, ob, si, so) in enumerate(bufs):
            c = 2 * g2 + b
            off = base + c * _CHUNK
            pltpu.make_async_copy(x_hbm.at[pl.ds(off, _CHUNK)], xb, si).wait()

            @pl.when(g2 > 0)
            def _wait_prev_out():
                pltpu.make_async_copy(
                    ob, out_hbm.at[pl.ds(off, _CHUNK)], so).wait()

            compute_chunk(xb, ob)
            pltpu.async_copy(ob, out_hbm.at[pl.ds(off, _CHUNK)], so)

            @pl.when(c + 2 < nch)
            def _start_next_in():
                pltpu.async_copy(
                    x_hbm.at[pl.ds(off + 2 * _CHUNK, _CHUNK)], xb, si)
        return 0

    lax.fori_loop(0, nch // 2, pair_body, 0)

    pltpu.make_async_copy(
        ob0, out_hbm.at[pl.ds(base + (nch - 2) * _CHUNK, _CHUNK)], so0).wait()
    pltpu.make_async_copy(
        ob1, out_hbm.at[pl.ds(base + (nch - 1) * _CHUNK, _CHUNK)], so1).wait()


def kernel(x, values):
    n_total = x.size
    mesh = plsc.VectorSubcoreMesh(core_axis_name="c", subcore_axis_name="s")
    vals_pad = jnp.pad(values, (0, 32 - NUM_KNOTS))
    run = functools.partial(
        pl.kernel,
        mesh=mesh,
        compiler_params=pltpu.CompilerParams(needs_layout_passes=False),
        out_type=jax.ShapeDtypeStruct((n_total,), jnp.float32),
        scratch_types=[
            pltpu.VMEM((32,), jnp.float32),
            pltpu.VMEM((32,), jnp.float32),
            pltpu.VMEM((32,), jnp.float32),
            pltpu.VMEM((32,), jnp.float32),
            pltpu.VMEM((32,), jnp.float32),
            pltpu.VMEM((_CHUNK,), jnp.float32),
            pltpu.VMEM((_CHUNK,), jnp.float32),
            pltpu.VMEM((_CHUNK,), jnp.float32),
            pltpu.VMEM((_CHUNK,), jnp.float32),
            pltpu.SemaphoreType.DMA,
            pltpu.SemaphoreType.DMA,
            pltpu.SemaphoreType.DMA,
            pltpu.SemaphoreType.DMA,
        ],
    )(_spline_body)
    out_flat = run(x.reshape(-1), vals_pad)
    return out_flat.reshape(x.shape)


# speedup vs baseline: 2.6000x; 2.1013x over previous
import functools

import jax
import jax.numpy as jnp
from jax import lax
from jax.experimental import pallas as pl
from jax.experimental.pallas import tpu as pltpu
from jax.experimental.pallas import tpu_sc as plsc

NUM_KNOTS = 20
X_MIN = -5.0
X_MAX = 5.0
_DX = (X_MAX - X_MIN) / (NUM_KNOTS - 1)

_NC = 2
_NS = 16
_NW = _NC * _NS
_L = 16

_CHUNK = 16384


_UNROLL = 4


def _spline_body(x_hbm, values_hbm, out_hbm, vals_v, t3, t2, t1, t0,
                 xb0, xb1, ob0, ob1, si0, si1, so0, so1):
    wid = lax.axis_index("s") * _NC + lax.axis_index("c")
    n_total = x_hbm.shape[0]
    per_w = n_total // _NW
    base = wid * per_w

    pltpu.sync_copy(values_hbm, vals_v)
    for t in range(2):
        k = lax.iota(jnp.int32, _L) + (t * _L)
        kc = jnp.minimum(k, NUM_KNOTS - 1)
        km1 = jnp.clip(k - 1, 0, NUM_KNOTS - 1)
        kp1 = jnp.minimum(k + 1, NUM_KNOTS - 1)
        kp2 = jnp.minimum(k + 2, NUM_KNOTS - 1)
        v0 = plsc.load_gather(vals_v, [km1])
        v1 = plsc.load_gather(vals_v, [kc])
        v2 = plsc.load_gather(vals_v, [kp1])
        v3 = plsc.load_gather(vals_v, [kp2])
        t3[pl.ds(t * _L, _L)] = 0.5 * (-v0 + 3.0 * v1 - 3.0 * v2 + v3)
        t2[pl.ds(t * _L, _L)] = 0.5 * (2.0 * v0 - 5.0 * v1 + 4.0 * v2 - v3)
        t1[pl.ds(t * _L, _L)] = 0.5 * (v2 - v0)
        t0[pl.ds(t * _L, _L)] = v1

    inv_dx = jnp.float32(1.0 / _DX)
    nch = per_w // _CHUNK

    def compute_chunk(xb, ob):
        @plsc.parallel_loop(0, _CHUNK // _L, 1, unroll=_UNROLL)
        def _vec_body(j):
            s = j * _L
            xv = xb[pl.ds(s, _L)]
            xc = jnp.minimum(jnp.maximum(xv, jnp.float32(X_MIN)),
                             jnp.float32(X_MAX))
            u = (xc - jnp.float32(X_MIN)) * inv_dx
            i = u.astype(jnp.int32)
            f = u - i.astype(jnp.float32)
            c3 = plsc.load_gather(t3, [i])
            c2 = plsc.load_gather(t2, [i])
            c1 = plsc.load_gather(t1, [i])
            c0 = plsc.load_gather(t0, [i])
            ob[pl.ds(s, _L)] = ((c3 * f + c2) * f + c1) * f + c0

    pltpu.async_copy(x_hbm.at[pl.ds(base, _CHUNK)], xb0, si0)
    pltpu.async_copy(x_hbm.at[pl.ds(base + _CHUNK, _CHUNK)], xb1, si1)

    bufs = ((xb0, ob0, si0, so0), (xb1, ob1, si1, so1))

    def pair_body(g2, _):
        for b, (xb, ob, si, so) in enumerate(bufs):
            c = 2 * g2 + b
            off = base + c * _CHUNK
            pltpu.make_async_copy(x_hbm.at[pl.ds(off, _CHUNK)], xb, si).wait()

            @pl.when(g2 > 0)
            def _wait_prev_out():
                pltpu.make_async_copy(
                    ob, out_hbm.at[pl.ds(off, _CHUNK)], so).wait()

            compute_chunk(xb, ob)
            pltpu.async_copy(ob, out_hbm.at[pl.ds(off, _CHUNK)], so)

            @pl.when(c + 2 < nch)
            def _start_next_in():
                pltpu.async_copy(
                    x_hbm.at[pl.ds(off + 2 * _CHUNK, _CHUNK)], xb, si)
        return 0

    lax.fori_loop(0, nch // 2, pair_body, 0)

    pltpu.make_async_copy(
        ob0, out_hbm.at[pl.ds(base + (nch - 2) * _CHUNK, _CHUNK)], so0).wait()
    pltpu.make_async_copy(
        ob1, out_hbm.at[pl.ds(base + (nch - 1) * _CHUNK, _CHUNK)], so1).wait()


def kernel(x, values):
    n_total = x.size
    mesh = plsc.VectorSubcoreMesh(core_axis_name="c", subcore_axis_name="s")
    vals_pad = jnp.pad(values, (0, 32 - NUM_KNOTS))
    run = functools.partial(
        pl.kernel,
        mesh=mesh,
        compiler_params=pltpu.CompilerParams(needs_layout_passes=False),
        out_type=jax.ShapeDtypeStruct((n_total,), jnp.float32),
        scratch_types=[
            pltpu.VMEM((32,), jnp.float32),
            pltpu.VMEM((32,), jnp.float32),
            pltpu.VMEM((32,), jnp.float32),
            pltpu.VMEM((32,), jnp.float32),
            pltpu.VMEM((32,), jnp.float32),
            pltpu.VMEM((_CHUNK,), jnp.float32),
            pltpu.VMEM((_CHUNK,), jnp.float32),
            pltpu.VMEM((_CHUNK,), jnp.float32),
            pltpu.VMEM((_CHUNK,), jnp.float32),
            pltpu.SemaphoreType.DMA,
            pltpu.SemaphoreType.DMA,
            pltpu.SemaphoreType.DMA,
            pltpu.SemaphoreType.DMA,
        ],
    )(_spline_body)
    out_flat = run(x.reshape(-1), vals_pad)
    return out_flat.reshape(x.shape)


# speedup vs baseline: 2.6803x; 1.0309x over previous
import functools

import jax
import jax.numpy as jnp
from jax import lax
from jax.experimental import pallas as pl
from jax.experimental.pallas import tpu as pltpu
from jax.experimental.pallas import tpu_sc as plsc

NUM_KNOTS = 20
X_MIN = -5.0
X_MAX = 5.0
_DX = (X_MAX - X_MIN) / (NUM_KNOTS - 1)

_NC = 2
_NS = 16
_NW = _NC * _NS
_L = 16

_CHUNK = 16384


_UNROLL = 8


def _spline_body(x_hbm, values_hbm, out_hbm, vals_v, t3, t2, t1, t0,
                 xb0, xb1, ob0, ob1, si0, si1, so0, so1):
    wid = lax.axis_index("s") * _NC + lax.axis_index("c")
    n_total = x_hbm.shape[0]
    per_w = n_total // _NW
    base = wid * per_w

    pltpu.sync_copy(values_hbm, vals_v)
    for t in range(2):
        k = lax.iota(jnp.int32, _L) + (t * _L)
        kc = jnp.minimum(k, NUM_KNOTS - 1)
        km1 = jnp.clip(k - 1, 0, NUM_KNOTS - 1)
        kp1 = jnp.minimum(k + 1, NUM_KNOTS - 1)
        kp2 = jnp.minimum(k + 2, NUM_KNOTS - 1)
        v0 = plsc.load_gather(vals_v, [km1])
        v1 = plsc.load_gather(vals_v, [kc])
        v2 = plsc.load_gather(vals_v, [kp1])
        v3 = plsc.load_gather(vals_v, [kp2])
        t3[pl.ds(t * _L, _L)] = 0.5 * (-v0 + 3.0 * v1 - 3.0 * v2 + v3)
        t2[pl.ds(t * _L, _L)] = 0.5 * (2.0 * v0 - 5.0 * v1 + 4.0 * v2 - v3)
        t1[pl.ds(t * _L, _L)] = 0.5 * (v2 - v0)
        t0[pl.ds(t * _L, _L)] = v1

    inv_dx = jnp.float32(1.0 / _DX)
    nch = per_w // _CHUNK

    def compute_chunk(xb, ob):
        @plsc.parallel_loop(0, _CHUNK // _L, 1, unroll=_UNROLL)
        def _vec_body(j):
            s = j * _L
            xv = xb[pl.ds(s, _L)]
            xc = jnp.minimum(jnp.maximum(xv, jnp.float32(X_MIN)),
                             jnp.float32(X_MAX))
            u = (xc - jnp.float32(X_MIN)) * inv_dx
            i = u.astype(jnp.int32)
            f = u - i.astype(jnp.float32)
            c3 = plsc.load_gather(t3, [i])
            c2 = plsc.load_gather(t2, [i])
            c1 = plsc.load_gather(t1, [i])
            c0 = plsc.load_gather(t0, [i])
            ob[pl.ds(s, _L)] = ((c3 * f + c2) * f + c1) * f + c0

    pltpu.async_copy(x_hbm.at[pl.ds(base, _CHUNK)], xb0, si0)
    pltpu.async_copy(x_hbm.at[pl.ds(base + _CHUNK, _CHUNK)], xb1, si1)

    bufs = ((xb0, ob0, si0, so0), (xb1, ob1, si1, so1))

    def pair_body(g2, _):
        for b, (xb, ob, si, so) in enumerate(bufs):
            c = 2 * g2 + b
            off = base + c * _CHUNK
            pltpu.make_async_copy(x_hbm.at[pl.ds(off, _CHUNK)], xb, si).wait()

            @pl.when(g2 > 0)
            def _wait_prev_out():
                pltpu.make_async_copy(
                    ob, out_hbm.at[pl.ds(off, _CHUNK)], so).wait()

            compute_chunk(xb, ob)
            pltpu.async_copy(ob, out_hbm.at[pl.ds(off, _CHUNK)], so)

            @pl.when(c + 2 < nch)
            def _start_next_in():
                pltpu.async_copy(
                    x_hbm.at[pl.ds(off + 2 * _CHUNK, _CHUNK)], xb, si)
        return 0

    lax.fori_loop(0, nch // 2, pair_body, 0)

    pltpu.make_async_copy(
        ob0, out_hbm.at[pl.ds(base + (nch - 2) * _CHUNK, _CHUNK)], so0).wait()
    pltpu.make_async_copy(
        ob1, out_hbm.at[pl.ds(base + (nch - 1) * _CHUNK, _CHUNK)], so1).wait()


def kernel(x, values):
    n_total = x.size
    mesh = plsc.VectorSubcoreMesh(core_axis_name="c", subcore_axis_name="s")
    vals_pad = jnp.pad(values, (0, 32 - NUM_KNOTS))
    run = functools.partial(
        pl.kernel,
        mesh=mesh,
        compiler_params=pltpu.CompilerParams(needs_layout_passes=False),
        out_type=jax.ShapeDtypeStruct((n_total,), jnp.float32),
        scratch_types=[
            pltpu.VMEM((32,), jnp.float32),
            pltpu.VMEM((32,), jnp.float32),
            pltpu.VMEM((32,), jnp.float32),
            pltpu.VMEM((32,), jnp.float32),
            pltpu.VMEM((32,), jnp.float32),
            pltpu.VMEM((_CHUNK,), jnp.float32),
            pltpu.VMEM((_CHUNK,), jnp.float32),
            pltpu.VMEM((_CHUNK,), jnp.float32),
            pltpu.VMEM((_CHUNK,), jnp.float32),
            pltpu.SemaphoreType.DMA,
            pltpu.SemaphoreType.DMA,
            pltpu.SemaphoreType.DMA,
            pltpu.SemaphoreType.DMA,
        ],
    )(_spline_body)
    out_flat = run(x.reshape(-1), vals_pad)
    return out_flat.reshape(x.shape)


# speedup vs baseline: 4.8970x; 1.8270x over previous
import functools

import jax
import jax.numpy as jnp
from jax import lax
from jax.experimental import pallas as pl
from jax.experimental.pallas import tpu as pltpu
from jax.experimental.pallas import tpu_sc as plsc

NUM_KNOTS = 20
X_MIN = -5.0
X_MAX = 5.0
_DX = (X_MAX - X_MIN) / (NUM_KNOTS - 1)

_NC = 2
_NS = 16
_NW = _NC * _NS
_L = 16

_BR = 8
_BC = 2048
_UNROLL = 2


def _spline_body(x_hbm, values_hbm, out_hbm, vals_v, t3, t2, t1, t0,
                 xb0, xb1, ob0, ob1, si0, si1, so0, so1):
    wid = lax.axis_index("s") * _NC + lax.axis_index("c")
    n_rows, n_cols = x_hbm.shape
    cblocks = n_cols // _BC
    rblocks_per_w = n_rows // (_BR * _NW)
    nblk = rblocks_per_w * cblocks
    row_base = wid * rblocks_per_w * _BR

    pltpu.sync_copy(values_hbm, vals_v)
    for t in range(2):
        k = lax.iota(jnp.int32, _L) + (t * _L)
        kc = jnp.minimum(k, NUM_KNOTS - 1)
        km1 = jnp.clip(k - 1, 0, NUM_KNOTS - 1)
        kp1 = jnp.minimum(k + 1, NUM_KNOTS - 1)
        kp2 = jnp.minimum(k + 2, NUM_KNOTS - 1)
        v0 = plsc.load_gather(vals_v, [km1])
        v1 = plsc.load_gather(vals_v, [kc])
        v2 = plsc.load_gather(vals_v, [kp1])
        v3 = plsc.load_gather(vals_v, [kp2])
        t3[pl.ds(t * _L, _L)] = 0.5 * (-v0 + 3.0 * v1 - 3.0 * v2 + v3)
        t2[pl.ds(t * _L, _L)] = 0.5 * (2.0 * v0 - 5.0 * v1 + 4.0 * v2 - v3)
        t1[pl.ds(t * _L, _L)] = 0.5 * (v2 - v0)
        t0[pl.ds(t * _L, _L)] = v1

    inv_dx = jnp.float32(1.0 / _DX)

    def blk_slice(b):
        rb = b // cblocks
        cb = b - rb * cblocks
        return (pl.ds(row_base + rb * _BR, _BR), pl.ds(cb * _BC, _BC))

    def compute_block(xb, ob):
        @plsc.parallel_loop(0, _BC // _L, 1, unroll=_UNROLL)
        def _vec_body(j):
            s = j * _L
            for r in range(_BR):
                xv = xb[r, pl.ds(s, _L)]
                xc = jnp.minimum(jnp.maximum(xv, jnp.float32(X_MIN)),
                                 jnp.float32(X_MAX))
                u = (xc - jnp.float32(X_MIN)) * inv_dx
                i = u.astype(jnp.int32)
                f = u - i.astype(jnp.float32)
                c3 = plsc.load_gather(t3, [i])
                c2 = plsc.load_gather(t2, [i])
                c1 = plsc.load_gather(t1, [i])
                c0 = plsc.load_gather(t0, [i])
                ob[r, pl.ds(s, _L)] = ((c3 * f + c2) * f + c1) * f + c0

    r0, c0_ = blk_slice(0)
    pltpu.async_copy(x_hbm.at[r0, c0_], xb0, si0)
    r1, c1_ = blk_slice(1)
    pltpu.async_copy(x_hbm.at[r1, c1_], xb1, si1)

    bufs = ((xb0, ob0, si0, so0), (xb1, ob1, si1, so1))

    def pair_body(g2, _):
        for b, (xb, ob, si, so) in enumerate(bufs):
            c = 2 * g2 + b
            rs, cs = blk_slice(c)
            pltpu.make_async_copy(x_hbm.at[rs, cs], xb, si).wait()

            @pl.when(g2 > 0)
            def _wait_prev_out():
                pltpu.make_async_copy(ob, out_hbm.at[rs, cs], so).wait()

            compute_block(xb, ob)
            pltpu.async_copy(ob, out_hbm.at[rs, cs], so)

            @pl.when(c + 2 < nblk)
            def _start_next_in():
                rs2, cs2 = blk_slice(c + 2)
                pltpu.async_copy(x_hbm.at[rs2, cs2], xb, si)
        return 0

    lax.fori_loop(0, nblk // 2, pair_body, 0)

    rs, cs = blk_slice(nblk - 2)
    pltpu.make_async_copy(ob0, out_hbm.at[rs, cs], so0).wait()
    rs, cs = blk_slice(nblk - 1)
    pltpu.make_async_copy(ob1, out_hbm.at[rs, cs], so1).wait()


def kernel(x, values):
    mesh = plsc.VectorSubcoreMesh(core_axis_name="c", subcore_axis_name="s")
    vals_pad = jnp.pad(values, (0, 32 - NUM_KNOTS))
    run = functools.partial(
        pl.kernel,
        mesh=mesh,
        compiler_params=pltpu.CompilerParams(needs_layout_passes=False),
        out_type=jax.ShapeDtypeStruct(x.shape, jnp.float32),
        scratch_types=[
            pltpu.VMEM((32,), jnp.float32),
            pltpu.VMEM((32,), jnp.float32),
            pltpu.VMEM((32,), jnp.float32),
            pltpu.VMEM((32,), jnp.float32),
            pltpu.VMEM((32,), jnp.float32),
            pltpu.VMEM((_BR, _BC), jnp.float32),
            pltpu.VMEM((_BR, _BC), jnp.float32),
            pltpu.VMEM((_BR, _BC), jnp.float32),
            pltpu.VMEM((_BR, _BC), jnp.float32),
            pltpu.SemaphoreType.DMA,
            pltpu.SemaphoreType.DMA,
            pltpu.SemaphoreType.DMA,
            pltpu.SemaphoreType.DMA,
        ],
    )(_spline_body)
    return run(x, vals_pad)


# speedup vs baseline: 5.0338x; 1.0279x over previous
import functools

import jax
import jax.numpy as jnp
from jax import lax
from jax.experimental import pallas as pl
from jax.experimental.pallas import tpu as pltpu
from jax.experimental.pallas import tpu_sc as plsc

NUM_KNOTS = 20
X_MIN = -5.0
X_MAX = 5.0
_DX = (X_MAX - X_MIN) / (NUM_KNOTS - 1)

_NC = 2
_NS = 16
_NW = _NC * _NS
_L = 16

_BR = 2
_BC = 8192
_UNROLL = 2


def _spline_body(x_hbm, values_hbm, out_hbm, vals_v, t3, t2, t1, t0,
                 xb0, xb1, ob0, ob1, si0, si1, so0, so1):
    wid = lax.axis_index("s") * _NC + lax.axis_index("c")
    n_rows, n_cols = x_hbm.shape
    cblocks = n_cols // _BC
    rblocks_per_w = n_rows // (_BR * _NW)
    nblk = rblocks_per_w * cblocks
    row_base = wid * rblocks_per_w * _BR

    pltpu.sync_copy(values_hbm, vals_v)
    for t in range(2):
        k = lax.iota(jnp.int32, _L) + (t * _L)
        kc = jnp.minimum(k, NUM_KNOTS - 1)
        km1 = jnp.clip(k - 1, 0, NUM_KNOTS - 1)
        kp1 = jnp.minimum(k + 1, NUM_KNOTS - 1)
        kp2 = jnp.minimum(k + 2, NUM_KNOTS - 1)
        v0 = plsc.load_gather(vals_v, [km1])
        v1 = plsc.load_gather(vals_v, [kc])
        v2 = plsc.load_gather(vals_v, [kp1])
        v3 = plsc.load_gather(vals_v, [kp2])
        t3[pl.ds(t * _L, _L)] = 0.5 * (-v0 + 3.0 * v1 - 3.0 * v2 + v3)
        t2[pl.ds(t * _L, _L)] = 0.5 * (2.0 * v0 - 5.0 * v1 + 4.0 * v2 - v3)
        t1[pl.ds(t * _L, _L)] = 0.5 * (v2 - v0)
        t0[pl.ds(t * _L, _L)] = v1

    inv_dx = jnp.float32(1.0 / _DX)

    def blk_slice(b):
        rb = b // cblocks
        cb = b - rb * cblocks
        return (pl.ds(row_base + rb * _BR, _BR), pl.ds(cb * _BC, _BC))

    def compute_block(xb, ob):
        @plsc.parallel_loop(0, _BC // _L, 1, unroll=_UNROLL)
        def _vec_body(j):
            s = j * _L
            for r in range(_BR):
                xv = xb[r, pl.ds(s, _L)]
                xc = jnp.minimum(jnp.maximum(xv, jnp.float32(X_MIN)),
                                 jnp.float32(X_MAX))
                u = (xc - jnp.float32(X_MIN)) * inv_dx
                i = u.astype(jnp.int32)
                f = u - i.astype(jnp.float32)
                c3 = plsc.load_gather(t3, [i])
                c2 = plsc.load_gather(t2, [i])
                c1 = plsc.load_gather(t1, [i])
                c0 = plsc.load_gather(t0, [i])
                ob[r, pl.ds(s, _L)] = ((c3 * f + c2) * f + c1) * f + c0

    r0, c0_ = blk_slice(0)
    pltpu.async_copy(x_hbm.at[r0, c0_], xb0, si0)
    r1, c1_ = blk_slice(1)
    pltpu.async_copy(x_hbm.at[r1, c1_], xb1, si1)

    bufs = ((xb0, ob0, si0, so0), (xb1, ob1, si1, so1))

    def pair_body(g2, _):
        for b, (xb, ob, si, so) in enumerate(bufs):
            c = 2 * g2 + b
            rs, cs = blk_slice(c)
            pltpu.make_async_copy(x_hbm.at[rs, cs], xb, si).wait()

            @pl.when(g2 > 0)
            def _wait_prev_out():
                pltpu.make_async_copy(ob, out_hbm.at[rs, cs], so).wait()

            compute_block(xb, ob)
            pltpu.async_copy(ob, out_hbm.at[rs, cs], so)

            @pl.when(c + 2 < nblk)
            def _start_next_in():
                rs2, cs2 = blk_slice(c + 2)
                pltpu.async_copy(x_hbm.at[rs2, cs2], xb, si)
        return 0

    lax.fori_loop(0, nblk // 2, pair_body, 0)

    rs, cs = blk_slice(nblk - 2)
    pltpu.make_async_copy(ob0, out_hbm.at[rs, cs], so0).wait()
    rs, cs = blk_slice(nblk - 1)
    pltpu.make_async_copy(ob1, out_hbm.at[rs, cs], so1).wait()


def kernel(x, values):
    mesh = plsc.VectorSubcoreMesh(core_axis_name="c", subcore_axis_name="s")
    vals_pad = jnp.pad(values, (0, 32 - NUM_KNOTS))
    run = functools.partial(
        pl.kernel,
        mesh=mesh,
        compiler_params=pltpu.CompilerParams(needs_layout_passes=False),
        out_type=jax.ShapeDtypeStruct(x.shape, jnp.float32),
        scratch_types=[
            pltpu.VMEM((32,), jnp.float32),
            pltpu.VMEM((32,), jnp.float32),
            pltpu.VMEM((32,), jnp.float32),
            pltpu.VMEM((32,), jnp.float32),
            pltpu.VMEM((32,), jnp.float32),
            pltpu.VMEM((_BR, _BC), jnp.float32),
            pltpu.VMEM((_BR, _BC), jnp.float32),
            pltpu.VMEM((_BR, _BC), jnp.float32),
            pltpu.VMEM((_BR, _BC), jnp.float32),
            pltpu.SemaphoreType.DMA,
            pltpu.SemaphoreType.DMA,
            pltpu.SemaphoreType.DMA,
            pltpu.SemaphoreType.DMA,
        ],
    )(_spline_body)
    return run(x, vals_pad)
